# baseline (speedup 1.0000x reference)
import jax
import jax.numpy as jnp
from jax.experimental import pallas as pl
from jax.experimental.pallas import tpu as pltpu
from jax.experimental.pallas import tpu_sc as plsc

_V = 100000
_D = 128
_B = 128
_STRIDE = 781


def _gather_body(table_hbm, out_hbm, sem):
    for i in range(_B):
        pltpu.make_async_copy(
            table_hbm.at[pl.ds(i * _STRIDE, 1)],
            out_hbm.at[pl.ds(i, 1)],
            sem,
        ).start()
    pltpu.make_async_copy(
        table_hbm.at[pl.ds(0, _B)],
        out_hbm,
        sem,
    ).wait()


def kernel(input):
    x = input.reshape(_V, _D)
    mesh = plsc.ScalarSubcoreMesh(axis_name="c", num_cores=1)
    k = pl.kernel(
        _gather_body,
        mesh=mesh,
        out_type=jax.ShapeDtypeStruct((_B, _D), jnp.float32),
        scratch_types=[
            pltpu.SemaphoreType.DMA,
        ],
    )
    return k(x)
